# baseline (speedup 1.0000x reference)
import functools

import jax
import jax.numpy as jnp
from jax import lax
from jax.experimental import pallas as pl
from jax.experimental.pallas import tpu as pltpu
from jax.experimental.pallas import tpu_sc as plsc

_NC = 2
_NS = 16
_NW = _NC * _NS
_L = 128


def _make_gather(S: int, T: int, D: int, K: int):
    assert S == _NW * _L and D == 64
    mesh = plsc.VectorSubcoreMesh(core_axis_name="c", subcore_axis_name="s")

    @functools.partial(
        pl.kernel,
        mesh=mesh,
        out_type=jax.ShapeDtypeStruct((T, 8, _NW, 8, _L), jnp.float32),
        scratch_types=[
            pltpu.VMEM((T, _L), jnp.int32),
            pltpu.VMEM((K, _L, D), jnp.float32),
            pltpu.VMEM((K, 8, 8, _L), jnp.float32),
            pltpu.SemaphoreType.DMA((K,)),
            pltpu.SemaphoreType.DMA((K,)),
        ],
        compiler_params=pltpu.CompilerParams(
            use_tc_tiling_on_sc=False, needs_layout_passes=False),
    )
    def gather_kernel(idxt_hbm, table_hbm, out_hbm, idx_v, rows_v, tr_v,
                      gsem, ssem):
        wid = lax.axis_index("s") * _NC + lax.axis_index("c")

        pltpu.sync_copy(idxt_hbm.at[:, pl.ds(wid * _L, _L)], idx_v)

        def gather(t, slot):
            return pltpu.make_async_copy(
                table_hbm.at[idx_v.at[t]], rows_v.at[slot], gsem.at[slot],
            )

        def store(t, slot):
            return pltpu.make_async_copy(
                tr_v.at[slot], out_hbm.at[t, :, wid], ssem.at[slot],
            )

        for t in range(K - 1):
            gather(t, t).start()

        iota16 = lax.iota(jnp.int32, 16)

        def body(t, carry):
            slot = lax.rem(t, K)
            gather(t, slot).wait()

            @pl.when(t >= K)
            def _():
                store(t - K, slot).wait()

            rows = rows_v.at[slot]
            tr = tr_v.at[slot]

            for d in range(D):
                col = jnp.full((16,), d, jnp.int32)
                for j in range(8):
                    v = plsc.load_gather(rows, [iota16 + (j * 16), col])
                    tr[d // 8, d % 8, pl.ds(j * 16, 16)] = v
            store(t, slot).start()

            @pl.when(t + K - 1 < T)
            def _():
                gather(t + K - 1, lax.rem(t + K - 1, K)).start()

            return carry

        lax.fori_loop(0, T, body, 0)

        for t in range(T - K, T):
            store(t, t % K).wait()

    return gather_kernel


def kernel(idx, pe):
    S, T = idx.shape
    V, D = pe.shape
    out5 = _make_gather(S, T, D, 3)(idx.T, pe)
    return out5.transpose(2, 4, 0, 1, 3).reshape(S, T, D)


# speedup vs baseline: 3.2983x; 3.2983x over previous
import functools

import jax
import jax.numpy as jnp
from jax import lax
from jax.experimental import pallas as pl
from jax.experimental.pallas import tpu as pltpu
from jax.experimental.pallas import tpu_sc as plsc

_NC = 2
_NS = 16
_NW = _NC * _NS
_L = 128


def _make_gather(R: int, D: int, K: int):
    assert R % _NW == 0
    n_chunks = R // _NW
    assert n_chunks >= K + 1
    mesh = plsc.VectorSubcoreMesh(core_axis_name="c", subcore_axis_name="s")

    @functools.partial(
        pl.kernel,
        mesh=mesh,
        out_type=jax.ShapeDtypeStruct((R, _L, _L), jnp.float32),
        scratch_types=[
            pltpu.VMEM((n_chunks, _L), jnp.int32),
            pltpu.VMEM((K, _L, D), jnp.float32),
            pltpu.SemaphoreType.DMA((K,)),
            pltpu.SemaphoreType.DMA((K,)),
        ],
        compiler_params=pltpu.CompilerParams(use_tc_tiling_on_sc=False),
    )
    def gather_kernel(idx_hbm, table_hbm, out_hbm, idx_v, rows_v, gsem, ssem):
        wid = lax.axis_index("s") * _NC + lax.axis_index("c")
        base = wid * n_chunks

        pltpu.sync_copy(idx_hbm.at[pl.ds(base, n_chunks)], idx_v)

        def gather(i, slot):
            return pltpu.make_async_copy(
                table_hbm.at[idx_v.at[i]],
                rows_v.at[slot], gsem.at[slot],
            )

        def store(i, slot):
            return pltpu.make_async_copy(
                rows_v.at[slot],
                out_hbm.at[base + i].at[:, pl.ds(0, D)],
                ssem.at[slot],
            )

        for i in range(K - 1):
            gather(i, i).start()

        gather(0, 0).wait()
        store(0, 0).start()
        gather(K - 1, K - 1).start()

        def body(i, carry):
            slot = lax.rem(i, K)
            gather(i, slot).wait()
            store(i, slot).start()
            ns = lax.rem(i + K - 1, K)
            store(i - 1, ns).wait()
            gather(i + K - 1, ns).start()
            return carry

        lax.fori_loop(1, n_chunks - K + 1, body, 0)

        for i in range(n_chunks - K + 1, n_chunks):
            gather(i, i % K).wait()
            store(i, i % K).start()
        for i in range(n_chunks - K, n_chunks):
            store(i, i % K).wait()

    return gather_kernel


def kernel(idx, pe):
    S, T = idx.shape
    V, D = pe.shape
    B = S * T
    idx_rows = idx.reshape(B // _L, _L)
    out_pad = _make_gather(B // _L, D, 10)(idx_rows, pe)
    return out_pad.reshape(S, T, _L)[:, :, :D]
